# baseline (speedup 1.0000x reference)
import functools

import jax
import jax.numpy as jnp
from jax import lax
from jax.experimental import pallas as pl
from jax.experimental.pallas import tpu as pltpu
from jax.experimental.pallas import tpu_sc as plsc

B = 4096
N_DENSE = 13
N_SPARSE = 26
VOCAB = 100000
D = 16

NUM_CORES = 2
NUM_SUBCORES = 16
NUM_WORKERS = NUM_CORES * NUM_SUBCORES
ROWS_PER_W = B // NUM_WORKERS
SLAB = ROWS_PER_W * N_SPARSE

LPG = 8
TABLE_LINES = N_SPARSE * VOCAB // LPG
ROWS_PER_CHUNK = 16
CHUNK = ROWS_PER_CHUNK * N_SPARSE
N_CHUNKS = ROWS_PER_W // ROWS_PER_CHUNK


def _sc_att(sparse_flat, offs, table):
    mesh = plsc.VectorSubcoreMesh(core_axis_name="c", subcore_axis_name="s")

    @functools.partial(
        pl.kernel,
        mesh=mesh,
        out_type=jax.ShapeDtypeStruct((B * D,), jnp.float32),
        compiler_params=pltpu.CompilerParams(needs_layout_passes=False),
        scratch_types=[
            pltpu.VMEM((SLAB,), jnp.int32),
            pltpu.VMEM((SLAB,), jnp.int32),
            pltpu.VMEM((CHUNK, 128), jnp.float32),
            pltpu.VMEM((CHUNK, 128), jnp.float32),
            pltpu.VMEM((ROWS_PER_W * D,), jnp.float32),
            pltpu.SemaphoreType.DMA,
            pltpu.SemaphoreType.DMA,
        ],
    )
    def body(sparse_hbm, offs_hbm, table_hbm, att_hbm,
             idx_v, m_v, g0, g1, att_v, sem0, sem1):
        cid = lax.axis_index("c")
        sid = lax.axis_index("s")
        wid = sid * NUM_CORES + cid
        base = wid * SLAB

        pltpu.sync_copy(sparse_hbm.at[pl.ds(base, SLAB)], idx_v)
        pltpu.sync_copy(offs_hbm, m_v)

        def prep(i, carry):
            sl = pl.ds(i * 16, 16)
            v = idx_v[sl]
            idx_v[sl] = m_v[sl] + lax.shift_right_logical(v, 3)
            m_v[sl] = lax.shift_left(lax.bitwise_and(v, 7), 4)
            return carry
        lax.fori_loop(0, SLAB // 16, prep, 0)

        def copy_desc(c, gbuf, sem):
            src = table_hbm.at[idx_v.at[pl.ds(c * CHUNK, CHUNK)]]
            return pltpu.make_async_copy(src, gbuf, sem)

        iota = lax.iota(jnp.int32, 16)

        def splat(x):
            return jnp.full((16,), x, jnp.int32)

        def reduce_chunk(c, gbuf):
            def row_body(r, carry):
                p0 = r * N_SPARSE
                gpos0 = c * CHUNK + p0
                mb = plsc.load_gather(m_v, [splat(gpos0)])
                v = plsc.load_gather(gbuf, [splat(p0), mb + iota])
                s_acc = v
                q_acc = v * v
                for f in range(1, N_SPARSE):
                    mb = plsc.load_gather(m_v, [splat(gpos0 + f)])
                    v = plsc.load_gather(gbuf, [splat(p0 + f), mb + iota])
                    s_acc = s_acc + v
                    q_acc = q_acc + v * v
                att = 0.5 * (s_acc * s_acc - q_acc)
                att_v[pl.ds((c * ROWS_PER_CHUNK + r) * D, D)] = att
                return carry
            lax.fori_loop(0, ROWS_PER_CHUNK, row_body, 0)

        copy_desc(0, g0, sem0).start()

        def pipe(i, carry):
            c0 = i * 2
            copy_desc(c0, g0, sem0).wait()
            copy_desc(c0 + 1, g1, sem1).start()
            reduce_chunk(c0, g0)
            copy_desc(c0 + 1, g1, sem1).wait()

            @pl.when(c0 + 2 < N_CHUNKS)
            def _():
                copy_desc(c0 + 2, g0, sem0).start()
            reduce_chunk(c0 + 1, g1)
            return carry
        lax.fori_loop(0, N_CHUNKS // 2, pipe, 0)

        pltpu.sync_copy(att_v, att_hbm.at[pl.ds(wid * ROWS_PER_W * D,
                                                ROWS_PER_W * D)])

    return body(sparse_flat, offs, table)


def _tc_head(att, wo_row, bo):
    def body(att_ref, wo_ref, bo_ref, out_ref):
        att_b = att_ref[...]
        wo = wo_ref[...]
        logit = jnp.sum(att_b * wo, axis=1, keepdims=True) + bo_ref[...]
        out_ref[...] = jax.nn.sigmoid(logit)

    return pl.pallas_call(
        body,
        out_shape=jax.ShapeDtypeStruct((B, 1), jnp.float32),
    )(att, wo_row, bo)


def kernel(inputs, emb_tables, W1, b1, W2, b2, Wo, bo):
    del W1, b1, W2, b2
    sparse_flat = inputs[:, N_DENSE:].reshape(-1)
    offs = (jnp.arange(SLAB, dtype=jnp.int32) % N_SPARSE) * (VOCAB // LPG)
    table = emb_tables.reshape(TABLE_LINES, 128)
    att = _sc_att(sparse_flat, offs, table).reshape(B, D)
    return _tc_head(att, Wo.reshape(1, D), bo.reshape(1, 1))


# speedup vs baseline: 8.3478x; 8.3478x over previous
import functools

import jax
import jax.numpy as jnp
from jax import lax
from jax.experimental import pallas as pl
from jax.experimental.pallas import tpu as pltpu
from jax.experimental.pallas import tpu_sc as plsc

B = 4096
N_DENSE = 13
N_SPARSE = 26
VOCAB = 100000
D = 16

NUM_CORES = 2
NUM_SUBCORES = 16
NUM_WORKERS = NUM_CORES * NUM_SUBCORES
N_ROWS = N_SPARSE * D
ROWS_PER_W = N_ROWS // NUM_WORKERS


def _sc_gather(vt_flat, table_t):
    mesh = plsc.VectorSubcoreMesh(core_axis_name="c", subcore_axis_name="s")

    @functools.partial(
        pl.kernel,
        mesh=mesh,
        out_type=jax.ShapeDtypeStruct((N_ROWS * B,), jnp.float32),
        compiler_params=pltpu.CompilerParams(needs_layout_passes=False),
        scratch_types=[
            pltpu.VMEM((VOCAB,), jnp.float32),
            pltpu.VMEM((B,), jnp.int32),
            pltpu.VMEM((B,), jnp.float32),
            pltpu.SemaphoreType.DMA,
        ],
    )
    def body(vt_hbm, table_hbm, x_hbm, row_v, idx_v, out_v, sem):
        cid = lax.axis_index("c")
        sid = lax.axis_index("s")
        wid = sid * NUM_CORES + cid
        r0 = wid * ROWS_PER_W

        def row_body(k, carry):
            r = r0 + k
            f = r // D
            pltpu.sync_copy(vt_hbm.at[pl.ds(f * B, B)], idx_v)
            pltpu.sync_copy(table_hbm.at[r], row_v)

            def gather16(i, carry2):
                sl = pl.ds(i * 16, 16)
                out_v[sl] = plsc.load_gather(row_v, [idx_v[sl]])
                return carry2
            lax.fori_loop(0, B // 16, gather16, 0)

            pltpu.sync_copy(out_v, x_hbm.at[pl.ds(r * B, B)])
            return carry
        lax.fori_loop(0, ROWS_PER_W, row_body, 0)

    return body(vt_flat, table_t)


def _tc_head(x, wo_col, bo):
    def body(x_ref, wo_ref, bo_ref, out_ref):
        xb = x_ref[...]
        s = jnp.sum(xb, axis=0)
        q = jnp.sum(xb * xb, axis=0)
        att = 0.5 * (s * s - q)
        logit = jnp.sum(att * wo_ref[...], axis=0, keepdims=True)
        out_ref[...] = jax.nn.sigmoid(logit + bo_ref[...])

    return pl.pallas_call(
        body,
        out_shape=jax.ShapeDtypeStruct((1, B), jnp.float32),
    )(x, wo_col, bo)


def kernel(inputs, emb_tables, W1, b1, W2, b2, Wo, bo):
    del W1, b1, W2, b2
    vt_flat = inputs[:, N_DENSE:].T.reshape(-1)
    table_t = jnp.transpose(emb_tables, (0, 2, 1)).reshape(N_ROWS, VOCAB)
    x = _sc_gather(vt_flat, table_t).reshape(N_SPARSE, D, B)
    out = _tc_head(x, Wo.reshape(D, 1), bo.reshape(1, 1))
    return out.reshape(B, 1)


# speedup vs baseline: 9.3527x; 1.1204x over previous
import functools

import jax
import jax.numpy as jnp
from jax import lax
from jax.experimental import pallas as pl
from jax.experimental.pallas import tpu as pltpu
from jax.experimental.pallas import tpu_sc as plsc

B = 4096
N_DENSE = 13
N_SPARSE = 26
VOCAB = 100000
D = 16

NUM_CORES = 2
NUM_SUBCORES = 16
NUM_WORKERS = NUM_CORES * NUM_SUBCORES
N_HALF = 2
F_PER_W = N_SPARSE // N_HALF


def _sc_gather(vt_flat, table_t):
    mesh = plsc.VectorSubcoreMesh(core_axis_name="c", subcore_axis_name="s")

    @functools.partial(
        pl.kernel,
        mesh=mesh,
        out_type=jax.ShapeDtypeStruct((2 * D * N_HALF * B,), jnp.float32),
        compiler_params=pltpu.CompilerParams(needs_layout_passes=False),
        scratch_types=[
            pltpu.VMEM((VOCAB,), jnp.float32),
            pltpu.VMEM((B,), jnp.int32),
            pltpu.VMEM((B,), jnp.float32),
            pltpu.VMEM((B,), jnp.float32),
            pltpu.SemaphoreType.DMA,
            pltpu.SemaphoreType.DMA,
        ],
    )
    def body(vt_hbm, table_hbm, p_hbm, row_v, idx_v, s_v, q_v, semr, semi):
        cid = lax.axis_index("c")
        sid = lax.axis_index("s")
        wid = sid * NUM_CORES + cid
        d = wid // N_HALF
        half = wid % N_HALF

        for k in range(F_PER_W):
            f = half * F_PER_W + k
            r = f * D + d
            rcp = pltpu.make_async_copy(table_hbm.at[r], row_v, semr)
            rcp.start()
            icp = pltpu.make_async_copy(vt_hbm.at[pl.ds(f * B, B)],
                                        idx_v, semi)
            icp.start()
            icp.wait()
            rcp.wait()

            if k == 0:
                def gather0(i, carry):
                    sl = pl.ds(i * 16, 16)
                    v = plsc.load_gather(row_v, [idx_v[sl]])
                    s_v[sl] = v
                    q_v[sl] = v * v
                    return carry
                lax.fori_loop(0, B // 16, gather0, 0)
            else:
                def gatheracc(i, carry):
                    sl = pl.ds(i * 16, 16)
                    v = plsc.load_gather(row_v, [idx_v[sl]])
                    s_v[sl] = s_v[sl] + v
                    q_v[sl] = q_v[sl] + v * v
                    return carry
                lax.fori_loop(0, B // 16, gatheracc, 0)

        off = (d * N_HALF + half) * B
        pltpu.sync_copy(s_v, p_hbm.at[pl.ds(off, B)])
        pltpu.sync_copy(q_v, p_hbm.at[pl.ds(D * N_HALF * B + off, B)])

    return body(vt_flat, table_t)


def _tc_head(p, wo_col, bo):
    def body(p_ref, wo_ref, bo_ref, out_ref):
        pb = p_ref[...]
        s = pb[0, :, 0, :] + pb[0, :, 1, :]
        q = pb[1, :, 0, :] + pb[1, :, 1, :]
        att = 0.5 * (s * s - q)
        logit = jnp.sum(att * wo_ref[...], axis=0, keepdims=True)
        out_ref[...] = jax.nn.sigmoid(logit + bo_ref[...])

    return pl.pallas_call(
        body,
        out_shape=jax.ShapeDtypeStruct((1, B), jnp.float32),
    )(p, wo_col, bo)


def kernel(inputs, emb_tables, W1, b1, W2, b2, Wo, bo):
    del W1, b1, W2, b2
    vt_flat = inputs[:, N_DENSE:].T.reshape(-1)
    table_t = jnp.transpose(emb_tables, (0, 2, 1)).reshape(N_SPARSE * D, VOCAB)
    p = _sc_gather(vt_flat, table_t).reshape(2, D, N_HALF, B)
    out = _tc_head(p, Wo.reshape(D, 1), bo.reshape(1, 1))
    return out.reshape(B, 1)
